# Initial kernel scaffold; baseline (speedup 1.0000x reference)
import jax
import jax.numpy as jnp
from jax.experimental import pallas as pl


def kernel(x, edge_index, batch, W1, b1, W2, b2, W3, b3, Wlin, blin):
    raise NotImplementedError("write your pallas kernel here")


# speedup vs baseline: 16.6075x; 16.6075x over previous
import functools

import jax
import jax.numpy as jnp
from jax import lax
from jax.experimental import pallas as pl
from jax.experimental.pallas import tpu as pltpu
from jax.experimental.pallas import tpu_sc as plsc

N = 10000
NG = 64
NC = 2
NS = 16
NACC = N + 8
RB = 624
RBW = 640
J = 8
EPAD = 327680
ROWS = EPAD // 128
BN = 1000
GB = N // BN


def _spmm_sc(u, src_r, dst_r, zeros, d, mode):
    mesh = plsc.VectorSubcoreMesh(core_axis_name="c", subcore_axis_name="s")
    rows_per_tile = ROWS // NS if mode == "feat" else ROWS // (NC * NS)
    n_groups = rows_per_tile // J

    def body(u_hbm, src_hbm, dst_hbm, zeros_hbm, out_hbm,
             idx_s, idx_d, rows, acc, sem):
        c = lax.axis_index("c")
        s = lax.axis_index("s")
        if mode == "deg":
            onev = jnp.ones((16,), jnp.float32)

            def fill(i, carry):
                for t in range(d // 16):
                    rows[i, pl.ds(t * 16, 16)] = onev
                return carry

            lax.fori_loop(0, 128, fill, 0)
        zstart = pl.multiple_of(s * RB, 8)
        pltpu.sync_copy(zeros_hbm.at[pl.ds(zstart, RBW)],
                        acc.at[pl.ds(zstart, RBW)])
        plsc.subcore_barrier()

        if mode == "feat":
            row0 = s * rows_per_tile
            off = lax.broadcast(c * jnp.int32(N), (16,))
        else:
            row0 = (c * NS + s) * rows_per_tile

        def group(g, carry):
            r = pl.multiple_of(row0 + g * J, 8)
            if mode != "deg":
                pltpu.sync_copy(src_hbm.at[pl.ds(r, J)], idx_s)
            pltpu.sync_copy(dst_hbm.at[pl.ds(r, J)], idx_d)
            if mode == "feat":
                for j in range(J):
                    for t in range(8):
                        idx_s[j, pl.ds(t * 16, 16)] = (
                            idx_s[j, pl.ds(t * 16, 16)] + off)
            for j in range(J):
                if mode != "deg":
                    pltpu.async_copy(u_hbm.at[idx_s.at[j]], rows, sem).wait()
                pltpu.sync_copy(rows, acc.at[idx_d.at[j]], add=True)
            return carry

        lax.fori_loop(0, n_groups, group, 0)
        plsc.subcore_barrier()
        rstart = pl.multiple_of(s * RB, 8)
        pltpu.sync_copy(acc.at[pl.ds(rstart, RBW)],
                        out_hbm.at[pl.ds(c * N + rstart, RBW)])

    return pl.kernel(
        body,
        out_type=jax.ShapeDtypeStruct((NC * N, d), jnp.float32),
        mesh=mesh,
        scratch_types=[
            pltpu.VMEM((J, 128), jnp.int32),
            pltpu.VMEM((J, 128), jnp.int32),
            pltpu.VMEM((128, d), jnp.float32),
            pltpu.VMEM_SHARED((NACC, d), jnp.float32),
            pltpu.SemaphoreType.DMA,
        ],
    )(u, src_r, dst_r, zeros)


def _tc_prep(degp, x):
    def body(dega, degb, x_ref, dinv_ref, u1_ref):
        d = dega[...][:, 0:1] + degb[...][:, 0:1] + 1.0
        dinv = lax.rsqrt(d)
        dinv_ref[...] = dinv
        u1_ref[...] = x_ref[...] * dinv

    return pl.pallas_call(
        body,
        grid=(GB,),
        in_specs=[
            pl.BlockSpec((BN, 16), lambda i: (i, 0)),
            pl.BlockSpec((BN, 16), lambda i: (GB + i, 0)),
            pl.BlockSpec((BN, 128), lambda i: (i, 0)),
        ],
        out_specs=[
            pl.BlockSpec((BN, 1), lambda i: (i, 0)),
            pl.BlockSpec((BN, 128), lambda i: (i, 0)),
        ],
        out_shape=[
            jax.ShapeDtypeStruct((N, 1), jnp.float32),
            jax.ShapeDtypeStruct((N, 128), jnp.float32),
        ],
    )(degp, degp, x)


def _tc_layer1(s1, u1, dinv, W1, b1, W2):
    def body(s1a, s1b, u1_ref, dinv_ref, W1_ref, b1_ref, W2_ref, u2_ref, h1_s):
        k = pl.program_id(1)

        @pl.when(k == 0)
        def _():
            z1 = dinv_ref[...] * (s1a[...] + s1b[...] + u1_ref[...])
            h1_s[...] = jnp.maximum(
                jnp.dot(z1, W1_ref[...], preferred_element_type=jnp.float32)
                + b1_ref[...], 0.0)

        u2_ref[...] = dinv_ref[...] * jnp.dot(
            h1_s[...], W2_ref[...], preferred_element_type=jnp.float32)

    return pl.pallas_call(
        body,
        grid=(GB, 2),
        in_specs=[
            pl.BlockSpec((BN, 128), lambda i, k: (i, 0)),
            pl.BlockSpec((BN, 128), lambda i, k: (GB + i, 0)),
            pl.BlockSpec((BN, 128), lambda i, k: (i, 0)),
            pl.BlockSpec((BN, 1), lambda i, k: (i, 0)),
            pl.BlockSpec((128, 256), lambda i, k: (0, 0)),
            pl.BlockSpec((1, 256), lambda i, k: (0, 0)),
            pl.BlockSpec((256, 128), lambda i, k: (0, k)),
        ],
        out_specs=pl.BlockSpec((BN, 128), lambda i, k: (k * GB + i, 0)),
        out_shape=jax.ShapeDtypeStruct((NC * N, 128), jnp.float32),
        scratch_shapes=[pltpu.VMEM((BN, 256), jnp.float32)],
    )(s1, s1, u1, dinv, W1, b1, W2)


def _tc_layer2(s2, u2, dinv, b2, W3):
    def body(s2a, s2b, u2a, u2b, dinv_ref, b2_ref, W3_ref, u3_ref):
        dv = dinv_ref[...]
        h2a = jnp.maximum(dv * (s2a[...] + u2a[...]) + b2_ref[...][:, 0:128],
                          0.0)
        h2b = jnp.maximum(dv * (s2b[...] + u2b[...]) + b2_ref[...][:, 128:256],
                          0.0)
        u3_ref[...] = dv * (
            jnp.dot(h2a, W3_ref[0:128, :], preferred_element_type=jnp.float32)
            + jnp.dot(h2b, W3_ref[128:256, :],
                      preferred_element_type=jnp.float32))

    return pl.pallas_call(
        body,
        grid=(GB,),
        in_specs=[
            pl.BlockSpec((BN, 128), lambda i: (i, 0)),
            pl.BlockSpec((BN, 128), lambda i: (GB + i, 0)),
            pl.BlockSpec((BN, 128), lambda i: (i, 0)),
            pl.BlockSpec((BN, 128), lambda i: (GB + i, 0)),
            pl.BlockSpec((BN, 1), lambda i: (i, 0)),
            pl.BlockSpec((1, 256), lambda i: (0, 0)),
            pl.BlockSpec((256, 128), lambda i: (0, 0)),
        ],
        out_specs=pl.BlockSpec((BN, 128), lambda i: (i, 0)),
        out_shape=jax.ShapeDtypeStruct((N, 128), jnp.float32),
    )(s2, s2, u2, u2, dinv, b2, W3)


def _tc_final(s3, u3, dinv, b3, batch2, Wlin, blin):
    def body(s3a, s3b, u3_ref, dinv_ref, b3_ref, batch_ref, Wlin_ref,
             blin_ref, out_ref, sums, cnts):
        i = pl.program_id(0)
        z3 = dinv_ref[...] * (s3a[...] + s3b[...] + u3_ref[...]) + b3_ref[...]
        g = batch_ref[...]
        iota = lax.broadcasted_iota(jnp.int32, (BN, NG), 1)
        oh = (g == iota).astype(jnp.float32)
        part = lax.dot_general(oh, z3, (((0,), (0,)), ((), ())),
                               preferred_element_type=jnp.float32)
        cnt = jnp.sum(oh, axis=0).reshape(NG, 1)

        @pl.when(i == 0)
        def _():
            sums[...] = part
            cnts[...] = cnt

        @pl.when(i > 0)
        def _():
            sums[...] += part
            cnts[...] += cnt

        @pl.when(i == GB - 1)
        def _():
            pooled = sums[...] / jnp.maximum(cnts[...], 1.0)
            out_ref[...] = jnp.dot(pooled, Wlin_ref[...],
                                   preferred_element_type=jnp.float32
                                   ) + blin_ref[...]

    return pl.pallas_call(
        body,
        grid=(GB,),
        in_specs=[
            pl.BlockSpec((BN, 128), lambda i: (i, 0)),
            pl.BlockSpec((BN, 128), lambda i: (GB + i, 0)),
            pl.BlockSpec((BN, 128), lambda i: (i, 0)),
            pl.BlockSpec((BN, 1), lambda i: (i, 0)),
            pl.BlockSpec((1, 128), lambda i: (0, 0)),
            pl.BlockSpec((BN, 1), lambda i: (i, 0)),
            pl.BlockSpec((128, 2), lambda i: (0, 0)),
            pl.BlockSpec((1, 2), lambda i: (0, 0)),
        ],
        out_specs=pl.BlockSpec((NG, 2), lambda i: (0, 0)),
        out_shape=jax.ShapeDtypeStruct((NG, 2), jnp.float32),
        scratch_shapes=[
            pltpu.VMEM((NG, 128), jnp.float32),
            pltpu.VMEM((NG, 1), jnp.float32),
        ],
    )(s3, s3, u3, dinv, b3, batch2, Wlin, blin)


def kernel(x, edge_index, batch, W1, b1, W2, b2, W3, b3, Wlin, blin):
    E = edge_index.shape[1]
    pad = EPAD - E
    src = edge_index[0]
    dst = edge_index[1]
    pad_src = (jnp.arange(pad, dtype=jnp.int32) * 37) % jnp.int32(N - 16)
    pad_dst = jnp.int32(N) + (jnp.arange(pad, dtype=jnp.int32) % 8)
    src_r = jnp.concatenate([src, pad_src]).reshape(ROWS, 128)
    dst_r = jnp.concatenate([dst, pad_dst]).reshape(ROWS, 128)

    zeros128 = jnp.zeros((N, 128), jnp.float32)
    zeros16 = jnp.zeros((N, 16), jnp.float32)
    ones16 = jnp.ones((N, 16), jnp.float32)

    b1r = b1.reshape(1, 256)
    b2r = b2.reshape(1, 256)
    b3r = b3.reshape(1, 128)
    blinr = blin.reshape(1, 2)
    batch2 = batch.reshape(N, 1)

    degp = _spmm_sc(ones16, src_r, dst_r, zeros16, 16, "deg")
    dinv, u1 = _tc_prep(degp, x)
    s1 = _spmm_sc(u1, src_r, dst_r, zeros128, 128, "edge")
    u2 = _tc_layer1(s1, u1, dinv, W1, b1r, W2)
    s2 = _spmm_sc(u2, src_r, dst_r, zeros128, 128, "feat")
    u3 = _tc_layer2(s2, u2, dinv, b2r, W3)
    s3 = _spmm_sc(u3, src_r, dst_r, zeros128, 128, "edge")
    return _tc_final(s3, u3, dinv, b3r, batch2, Wlin, blinr)


# speedup vs baseline: 23.2546x; 1.4003x over previous
import functools

import jax
import jax.numpy as jnp
from jax import lax
from jax.experimental import pallas as pl
from jax.experimental.pallas import tpu as pltpu
from jax.experimental.pallas import tpu_sc as plsc

N = 10000
NG = 64
NC = 2
NS = 16
NACC = N + 8
RB = 624
RBW = 640
J = 8
EPAD = 327680
ROWS = EPAD // 128
BN = 1000
GB = N // BN


def _spmm_sc(u, src_r, dst_r, zeros, d, mode):
    mesh = plsc.VectorSubcoreMesh(core_axis_name="c", subcore_axis_name="s")
    rows_per_tile = ROWS // NS if mode == "feat" else ROWS // (NC * NS)
    n_pairs = rows_per_tile // (2 * J)
    NB = 2

    def body(u_hbm, src_hbm, dst_hbm, zeros_hbm, out_hbm,
             idx_s, idx_da, idx_db, rows, acc, s0, s1):
        c = lax.axis_index("c")
        s = lax.axis_index("s")
        sems = [s0, s1]
        if mode == "deg":
            onev = jnp.ones((16,), jnp.float32)

            def fill(i, carry):
                for t in range(d // 16):
                    rows[0, i, pl.ds(t * 16, 16)] = onev
                return carry

            lax.fori_loop(0, 128, fill, 0)
        zstart = pl.multiple_of(s * RB, 8)
        pltpu.sync_copy(zeros_hbm.at[pl.ds(zstart, RBW)],
                        acc.at[pl.ds(zstart, RBW)])
        plsc.subcore_barrier()

        if mode == "feat":
            row0 = s * rows_per_tile
            off = lax.broadcast(c * jnp.int32(N), (16,))
        else:
            row0 = (c * NS + s) * rows_per_tile

        def load_idx(r, idx_d):
            if mode != "deg":
                pltpu.sync_copy(src_hbm.at[pl.ds(r, J)], idx_s)
            pltpu.sync_copy(dst_hbm.at[pl.ds(r, J)], idx_d)
            if mode == "feat":
                for j in range(J):
                    for t in range(8):
                        idx_s[j, pl.ds(t * 16, 16)] = (
                            idx_s[j, pl.ds(t * 16, 16)] + off)

        def scat(idx_d, b, buf):
            return pltpu.async_copy(rows.at[buf], acc.at[idx_d.at[b]],
                                    sems[buf], add=True)

        if mode == "deg":
            def pair(g2, carry):
                for idx_d, dr in ((idx_da, 0), (idx_db, J)):
                    r = pl.multiple_of(row0 + g2 * 2 * J + dr, 8)
                    pltpu.sync_copy(dst_hbm.at[pl.ds(r, J)], idx_d)
                    sds = [scat(idx_d, b, 0) for b in range(J)]
                    for sd in sds:
                        sd.wait()
                return carry

            lax.fori_loop(0, n_pairs, pair, 0)
        else:
            def half_run(idx_d, idx_d_prev, first, g2):
                gds = [None] * J
                sds = [None] * J
                for b in range(J):
                    buf = b % NB
                    if b < NB:
                        if first:
                            @pl.when(g2 > 0)
                            def _(b=b, buf=buf):
                                pltpu.make_async_copy(
                                    rows.at[buf],
                                    acc.at[idx_d_prev.at[J - NB + b]],
                                    sems[buf]).wait()
                        else:
                            idx_d_prev[J - NB + b].wait()
                    else:
                        sds[b - NB].wait()
                    gds[b] = pltpu.async_copy(u_hbm.at[idx_s.at[b]],
                                              rows.at[buf], sems[buf])
                    if b >= 1:
                        gds[b - 1].wait()
                        sds[b - 1] = scat(idx_d, b - 1, (b - 1) % NB)
                gds[J - 1].wait()
                sds[J - 1] = scat(idx_d, J - 1, (J - 1) % NB)
                return sds

            def pair(g2, carry):
                r = pl.multiple_of(row0 + g2 * 2 * J, 8)
                load_idx(r, idx_da)
                sds_a = half_run(idx_da, idx_db, True, g2)
                load_idx(r + J, idx_db)
                half_run(idx_db, sds_a, False, g2)
                return carry

            lax.fori_loop(0, n_pairs, pair, 0)
            for b in range(J - NB, J):
                pltpu.make_async_copy(rows.at[b % NB],
                                      acc.at[idx_db.at[b]],
                                      sems[b % NB]).wait()

        plsc.subcore_barrier()
        rstart = pl.multiple_of(s * RB, 8)
        pltpu.sync_copy(acc.at[pl.ds(rstart, RBW)],
                        out_hbm.at[pl.ds(c * N + rstart, RBW)])

    return pl.kernel(
        body,
        out_type=jax.ShapeDtypeStruct((NC * N, d), jnp.float32),
        mesh=mesh,
        scratch_types=[
            pltpu.VMEM((J, 128), jnp.int32),
            pltpu.VMEM((J, 128), jnp.int32),
            pltpu.VMEM((J, 128), jnp.int32),
            pltpu.VMEM((NB, 128, d), jnp.float32),
            pltpu.VMEM_SHARED((NACC, d), jnp.float32),
            pltpu.SemaphoreType.DMA,
            pltpu.SemaphoreType.DMA,
        ],
    )(u, src_r, dst_r, zeros)


def _tc_prep(degp, x):
    def body(dega, degb, x_ref, dinv_ref, u1_ref):
        d = dega[...][:, 0:1] + degb[...][:, 0:1] + 1.0
        dinv = lax.rsqrt(d)
        dinv_ref[...] = dinv
        u1_ref[...] = x_ref[...] * dinv

    return pl.pallas_call(
        body,
        grid=(GB,),
        in_specs=[
            pl.BlockSpec((BN, 16), lambda i: (i, 0)),
            pl.BlockSpec((BN, 16), lambda i: (GB + i, 0)),
            pl.BlockSpec((BN, 128), lambda i: (i, 0)),
        ],
        out_specs=[
            pl.BlockSpec((BN, 1), lambda i: (i, 0)),
            pl.BlockSpec((BN, 128), lambda i: (i, 0)),
        ],
        out_shape=[
            jax.ShapeDtypeStruct((N, 1), jnp.float32),
            jax.ShapeDtypeStruct((N, 128), jnp.float32),
        ],
    )(degp, degp, x)


def _tc_layer1(s1, u1, dinv, W1, b1, W2):
    def body(s1a, s1b, u1_ref, dinv_ref, W1_ref, b1_ref, W2_ref, u2_ref, h1_s):
        k = pl.program_id(1)

        @pl.when(k == 0)
        def _():
            z1 = dinv_ref[...] * (s1a[...] + s1b[...] + u1_ref[...])
            h1_s[...] = jnp.maximum(
                jnp.dot(z1, W1_ref[...], preferred_element_type=jnp.float32)
                + b1_ref[...], 0.0)

        u2_ref[...] = dinv_ref[...] * jnp.dot(
            h1_s[...], W2_ref[...], preferred_element_type=jnp.float32)

    return pl.pallas_call(
        body,
        grid=(GB, 2),
        in_specs=[
            pl.BlockSpec((BN, 128), lambda i, k: (i, 0)),
            pl.BlockSpec((BN, 128), lambda i, k: (GB + i, 0)),
            pl.BlockSpec((BN, 128), lambda i, k: (i, 0)),
            pl.BlockSpec((BN, 1), lambda i, k: (i, 0)),
            pl.BlockSpec((128, 256), lambda i, k: (0, 0)),
            pl.BlockSpec((1, 256), lambda i, k: (0, 0)),
            pl.BlockSpec((256, 128), lambda i, k: (0, k)),
        ],
        out_specs=pl.BlockSpec((BN, 128), lambda i, k: (k * GB + i, 0)),
        out_shape=jax.ShapeDtypeStruct((NC * N, 128), jnp.float32),
        scratch_shapes=[pltpu.VMEM((BN, 256), jnp.float32)],
    )(s1, s1, u1, dinv, W1, b1, W2)


def _tc_layer2(s2, u2, dinv, b2, W3):
    def body(s2a, s2b, u2a, u2b, dinv_ref, b2_ref, W3_ref, u3_ref):
        dv = dinv_ref[...]
        h2a = jnp.maximum(dv * (s2a[...] + u2a[...]) + b2_ref[...][:, 0:128],
                          0.0)
        h2b = jnp.maximum(dv * (s2b[...] + u2b[...]) + b2_ref[...][:, 128:256],
                          0.0)
        u3_ref[...] = dv * (
            jnp.dot(h2a, W3_ref[0:128, :], preferred_element_type=jnp.float32)
            + jnp.dot(h2b, W3_ref[128:256, :],
                      preferred_element_type=jnp.float32))

    return pl.pallas_call(
        body,
        grid=(GB,),
        in_specs=[
            pl.BlockSpec((BN, 128), lambda i: (i, 0)),
            pl.BlockSpec((BN, 128), lambda i: (GB + i, 0)),
            pl.BlockSpec((BN, 128), lambda i: (i, 0)),
            pl.BlockSpec((BN, 128), lambda i: (GB + i, 0)),
            pl.BlockSpec((BN, 1), lambda i: (i, 0)),
            pl.BlockSpec((1, 256), lambda i: (0, 0)),
            pl.BlockSpec((256, 128), lambda i: (0, 0)),
        ],
        out_specs=pl.BlockSpec((BN, 128), lambda i: (i, 0)),
        out_shape=jax.ShapeDtypeStruct((N, 128), jnp.float32),
    )(s2, s2, u2, u2, dinv, b2, W3)


def _tc_final(s3, u3, dinv, b3, batch2, Wlin, blin):
    def body(s3a, s3b, u3_ref, dinv_ref, b3_ref, batch_ref, Wlin_ref,
             blin_ref, out_ref, sums, cnts):
        i = pl.program_id(0)
        z3 = dinv_ref[...] * (s3a[...] + s3b[...] + u3_ref[...]) + b3_ref[...]
        g = batch_ref[...]
        iota = lax.broadcasted_iota(jnp.int32, (BN, NG), 1)
        oh = (g == iota).astype(jnp.float32)
        part = lax.dot_general(oh, z3, (((0,), (0,)), ((), ())),
                               preferred_element_type=jnp.float32)
        cnt = jnp.sum(oh, axis=0).reshape(NG, 1)

        @pl.when(i == 0)
        def _():
            sums[...] = part
            cnts[...] = cnt

        @pl.when(i > 0)
        def _():
            sums[...] += part
            cnts[...] += cnt

        @pl.when(i == GB - 1)
        def _():
            pooled = sums[...] / jnp.maximum(cnts[...], 1.0)
            out_ref[...] = jnp.dot(pooled, Wlin_ref[...],
                                   preferred_element_type=jnp.float32
                                   ) + blin_ref[...]

    return pl.pallas_call(
        body,
        grid=(GB,),
        in_specs=[
            pl.BlockSpec((BN, 128), lambda i: (i, 0)),
            pl.BlockSpec((BN, 128), lambda i: (GB + i, 0)),
            pl.BlockSpec((BN, 128), lambda i: (i, 0)),
            pl.BlockSpec((BN, 1), lambda i: (i, 0)),
            pl.BlockSpec((1, 128), lambda i: (0, 0)),
            pl.BlockSpec((BN, 1), lambda i: (i, 0)),
            pl.BlockSpec((128, 2), lambda i: (0, 0)),
            pl.BlockSpec((1, 2), lambda i: (0, 0)),
        ],
        out_specs=pl.BlockSpec((NG, 2), lambda i: (0, 0)),
        out_shape=jax.ShapeDtypeStruct((NG, 2), jnp.float32),
        scratch_shapes=[
            pltpu.VMEM((NG, 128), jnp.float32),
            pltpu.VMEM((NG, 1), jnp.float32),
        ],
    )(s3, s3, u3, dinv, b3, batch2, Wlin, blin)


def kernel(x, edge_index, batch, W1, b1, W2, b2, W3, b3, Wlin, blin):
    E = edge_index.shape[1]
    pad = EPAD - E
    src = edge_index[0]
    dst = edge_index[1]
    pad_src = (jnp.arange(pad, dtype=jnp.int32) * 37) % jnp.int32(N - 16)
    pad_dst = jnp.int32(N) + (jnp.arange(pad, dtype=jnp.int32) % 8)
    src_r = jnp.concatenate([src, pad_src]).reshape(ROWS, 128)
    dst_r = jnp.concatenate([dst, pad_dst]).reshape(ROWS, 128)

    zeros128 = jnp.zeros((N, 128), jnp.float32)
    zeros16 = jnp.zeros((N, 16), jnp.float32)
    ones16 = jnp.ones((N, 16), jnp.float32)

    b1r = b1.reshape(1, 256)
    b2r = b2.reshape(1, 256)
    b3r = b3.reshape(1, 128)
    blinr = blin.reshape(1, 2)
    batch2 = batch.reshape(N, 1)

    degp = _spmm_sc(ones16, src_r, dst_r, zeros16, 16, "deg")
    dinv, u1 = _tc_prep(degp, x)
    s1 = _spmm_sc(u1, src_r, dst_r, zeros128, 128, "edge")
    u2 = _tc_layer1(s1, u1, dinv, W1, b1r, W2)
    s2 = _spmm_sc(u2, src_r, dst_r, zeros128, 128, "feat")
    u3 = _tc_layer2(s2, u2, dinv, b2r, W3)
    s3 = _spmm_sc(u3, src_r, dst_r, zeros128, 128, "edge")
    return _tc_final(s3, u3, dinv, b3r, batch2, Wlin, blinr)


# speedup vs baseline: 23.3058x; 1.0022x over previous
import functools

import jax
import jax.numpy as jnp
from jax import lax
from jax.experimental import pallas as pl
from jax.experimental.pallas import tpu as pltpu
from jax.experimental.pallas import tpu_sc as plsc

N = 10000
NG = 64
NC = 2
NS = 16
NACC = N + 8
RB = 624
RBW = 640
SW = 64
SPS = 16
EPAD = 327680
ROWS = EPAD // SW
BN = 1000
GB = N // BN


def _spmm_sc(u, src_r, dst_r, zeros, d, mode):
    mesh = plsc.VectorSubcoreMesh(core_axis_name="c", subcore_axis_name="s")
    slabs_per_tile = (ROWS // NS if mode == "feat" else ROWS // (NC * NS)) // SPS
    n_pairs = slabs_per_tile // 2
    NB = 4
    LAG = 2

    def body(u_hbm, src_hbm, dst_hbm, zeros_hbm, out_hbm,
             idx_s, idx_da, idx_db, rows, acc, s0, s1, s2, s3):
        c = lax.axis_index("c")
        s = lax.axis_index("s")
        sems = [s0, s1, s2, s3]
        if mode == "deg":
            onev = jnp.ones((16,), jnp.float32)

            def fill(i, carry):
                for t in range(d // 16):
                    rows[0, i, pl.ds(t * 16, 16)] = onev
                return carry

            lax.fori_loop(0, SW, fill, 0)
        zstart = pl.multiple_of(s * RB, 8)
        pltpu.sync_copy(zeros_hbm.at[pl.ds(zstart, RBW)],
                        acc.at[pl.ds(zstart, RBW)])
        plsc.subcore_barrier()

        if mode == "feat":
            row0 = s * slabs_per_tile * SPS
            off = lax.broadcast(c * jnp.int32(N), (16,))
        else:
            row0 = (c * NS + s) * slabs_per_tile * SPS

        def load_idx(r, idx_d):
            if mode != "deg":
                pltpu.sync_copy(src_hbm.at[pl.ds(r, SPS)], idx_s)
            pltpu.sync_copy(dst_hbm.at[pl.ds(r, SPS)], idx_d)
            if mode == "feat":
                for j in range(SPS):
                    for t in range(SW // 16):
                        idx_s[j, pl.ds(t * 16, 16)] = (
                            idx_s[j, pl.ds(t * 16, 16)] + off)

        def scat(idx_d, b, buf):
            return pltpu.async_copy(rows.at[buf], acc.at[idx_d.at[b]],
                                    sems[buf], add=True)

        if mode == "deg":
            def pair(g2, carry):
                for idx_d, dr in ((idx_da, 0), (idx_db, SPS)):
                    r = pl.multiple_of(row0 + g2 * 2 * SPS + dr, 8)
                    pltpu.sync_copy(dst_hbm.at[pl.ds(r, SPS)], idx_d)
                    sds = [scat(idx_d, b, 0) for b in range(SPS)]
                    for sd in sds:
                        sd.wait()
                return carry

            lax.fori_loop(0, n_pairs, pair, 0)
        else:
            def half_run(idx_d, idx_d_prev, first, g2):
                gds = [None] * SPS
                sds = [None] * SPS
                for b in range(SPS):
                    buf = b % NB
                    if b < NB:
                        if first:
                            @pl.when(g2 > 0)
                            def _(b=b, buf=buf):
                                pltpu.make_async_copy(
                                    rows.at[buf],
                                    acc.at[idx_d_prev.at[SPS - NB + b]],
                                    sems[buf]).wait()
                        else:
                            idx_d_prev[SPS - NB + b].wait()
                    else:
                        sds[b - NB].wait()
                    gds[b] = pltpu.async_copy(u_hbm.at[idx_s.at[b]],
                                              rows.at[buf], sems[buf])
                    if b >= LAG:
                        gds[b - LAG].wait()
                        sds[b - LAG] = scat(idx_d, b - LAG, (b - LAG) % NB)
                for b in range(SPS - LAG, SPS):
                    gds[b].wait()
                    sds[b] = scat(idx_d, b, b % NB)
                return sds

            def pair(g2, carry):
                r = pl.multiple_of(row0 + g2 * 2 * SPS, 8)
                load_idx(r, idx_da)
                sds_a = half_run(idx_da, idx_db, True, g2)
                load_idx(r + SPS, idx_db)
                half_run(idx_db, sds_a, False, g2)
                return carry

            lax.fori_loop(0, n_pairs, pair, 0)
            for b in range(SPS - NB, SPS):
                pltpu.make_async_copy(rows.at[b % NB],
                                      acc.at[idx_db.at[b]],
                                      sems[b % NB]).wait()

        plsc.subcore_barrier()
        rstart = pl.multiple_of(s * RB, 8)
        pltpu.sync_copy(acc.at[pl.ds(rstart, RBW)],
                        out_hbm.at[pl.ds(c * N + rstart, RBW)])

    return pl.kernel(
        body,
        out_type=jax.ShapeDtypeStruct((NC * N, d), jnp.float32),
        mesh=mesh,
        scratch_types=[
            pltpu.VMEM((SPS, SW), jnp.int32),
            pltpu.VMEM((SPS, SW), jnp.int32),
            pltpu.VMEM((SPS, SW), jnp.int32),
            pltpu.VMEM((NB, SW, d), jnp.float32),
            pltpu.VMEM_SHARED((NACC, d), jnp.float32),
            pltpu.SemaphoreType.DMA,
            pltpu.SemaphoreType.DMA,
            pltpu.SemaphoreType.DMA,
            pltpu.SemaphoreType.DMA,
        ],
    )(u, src_r, dst_r, zeros)


def _tc_prep(degp, x):
    def body(dega, degb, x_ref, dinv_ref, u1_ref):
        d = dega[...][:, 0:1] + degb[...][:, 0:1] + 1.0
        dinv = lax.rsqrt(d)
        dinv_ref[...] = dinv
        u1_ref[...] = x_ref[...] * dinv

    return pl.pallas_call(
        body,
        grid=(GB,),
        in_specs=[
            pl.BlockSpec((BN, 16), lambda i: (i, 0)),
            pl.BlockSpec((BN, 16), lambda i: (GB + i, 0)),
            pl.BlockSpec((BN, 128), lambda i: (i, 0)),
        ],
        out_specs=[
            pl.BlockSpec((BN, 1), lambda i: (i, 0)),
            pl.BlockSpec((BN, 128), lambda i: (i, 0)),
        ],
        out_shape=[
            jax.ShapeDtypeStruct((N, 1), jnp.float32),
            jax.ShapeDtypeStruct((N, 128), jnp.float32),
        ],
    )(degp, degp, x)


def _tc_layer1(s1, u1, dinv, W1, b1, W2):
    def body(s1a, s1b, u1_ref, dinv_ref, W1_ref, b1_ref, W2_ref, u2_ref, h1_s):
        k = pl.program_id(1)

        @pl.when(k == 0)
        def _():
            z1 = dinv_ref[...] * (s1a[...] + s1b[...] + u1_ref[...])
            h1_s[...] = jnp.maximum(
                jnp.dot(z1, W1_ref[...], preferred_element_type=jnp.float32)
                + b1_ref[...], 0.0)

        u2_ref[...] = dinv_ref[...] * jnp.dot(
            h1_s[...], W2_ref[...], preferred_element_type=jnp.float32)

    return pl.pallas_call(
        body,
        grid=(GB, 2),
        in_specs=[
            pl.BlockSpec((BN, 128), lambda i, k: (i, 0)),
            pl.BlockSpec((BN, 128), lambda i, k: (GB + i, 0)),
            pl.BlockSpec((BN, 128), lambda i, k: (i, 0)),
            pl.BlockSpec((BN, 1), lambda i, k: (i, 0)),
            pl.BlockSpec((128, 256), lambda i, k: (0, 0)),
            pl.BlockSpec((1, 256), lambda i, k: (0, 0)),
            pl.BlockSpec((256, 128), lambda i, k: (0, k)),
        ],
        out_specs=pl.BlockSpec((BN, 128), lambda i, k: (k * GB + i, 0)),
        out_shape=jax.ShapeDtypeStruct((NC * N, 128), jnp.float32),
        scratch_shapes=[pltpu.VMEM((BN, 256), jnp.float32)],
    )(s1, s1, u1, dinv, W1, b1, W2)


def _tc_layer2(s2, u2, dinv, b2, W3):
    def body(s2a, s2b, u2a, u2b, dinv_ref, b2_ref, W3_ref, u3_ref):
        dv = dinv_ref[...]
        h2a = jnp.maximum(dv * (s2a[...] + u2a[...]) + b2_ref[...][:, 0:128],
                          0.0)
        h2b = jnp.maximum(dv * (s2b[...] + u2b[...]) + b2_ref[...][:, 128:256],
                          0.0)
        u3_ref[...] = dv * (
            jnp.dot(h2a, W3_ref[0:128, :], preferred_element_type=jnp.float32)
            + jnp.dot(h2b, W3_ref[128:256, :],
                      preferred_element_type=jnp.float32))

    return pl.pallas_call(
        body,
        grid=(GB,),
        in_specs=[
            pl.BlockSpec((BN, 128), lambda i: (i, 0)),
            pl.BlockSpec((BN, 128), lambda i: (GB + i, 0)),
            pl.BlockSpec((BN, 128), lambda i: (i, 0)),
            pl.BlockSpec((BN, 128), lambda i: (GB + i, 0)),
            pl.BlockSpec((BN, 1), lambda i: (i, 0)),
            pl.BlockSpec((1, 256), lambda i: (0, 0)),
            pl.BlockSpec((256, 128), lambda i: (0, 0)),
        ],
        out_specs=pl.BlockSpec((BN, 128), lambda i: (i, 0)),
        out_shape=jax.ShapeDtypeStruct((N, 128), jnp.float32),
    )(s2, s2, u2, u2, dinv, b2, W3)


def _tc_final(s3, u3, dinv, b3, batch2, Wlin, blin):
    def body(s3a, s3b, u3_ref, dinv_ref, b3_ref, batch_ref, Wlin_ref,
             blin_ref, out_ref, sums, cnts):
        i = pl.program_id(0)
        z3 = dinv_ref[...] * (s3a[...] + s3b[...] + u3_ref[...]) + b3_ref[...]
        g = batch_ref[...]
        iota = lax.broadcasted_iota(jnp.int32, (BN, NG), 1)
        oh = (g == iota).astype(jnp.float32)
        part = lax.dot_general(oh, z3, (((0,), (0,)), ((), ())),
                               preferred_element_type=jnp.float32)
        cnt = jnp.sum(oh, axis=0).reshape(NG, 1)

        @pl.when(i == 0)
        def _():
            sums[...] = part
            cnts[...] = cnt

        @pl.when(i > 0)
        def _():
            sums[...] += part
            cnts[...] += cnt

        @pl.when(i == GB - 1)
        def _():
            pooled = sums[...] / jnp.maximum(cnts[...], 1.0)
            out_ref[...] = jnp.dot(pooled, Wlin_ref[...],
                                   preferred_element_type=jnp.float32
                                   ) + blin_ref[...]

    return pl.pallas_call(
        body,
        grid=(GB,),
        in_specs=[
            pl.BlockSpec((BN, 128), lambda i: (i, 0)),
            pl.BlockSpec((BN, 128), lambda i: (GB + i, 0)),
            pl.BlockSpec((BN, 128), lambda i: (i, 0)),
            pl.BlockSpec((BN, 1), lambda i: (i, 0)),
            pl.BlockSpec((1, 128), lambda i: (0, 0)),
            pl.BlockSpec((BN, 1), lambda i: (i, 0)),
            pl.BlockSpec((128, 2), lambda i: (0, 0)),
            pl.BlockSpec((1, 2), lambda i: (0, 0)),
        ],
        out_specs=pl.BlockSpec((NG, 2), lambda i: (0, 0)),
        out_shape=jax.ShapeDtypeStruct((NG, 2), jnp.float32),
        scratch_shapes=[
            pltpu.VMEM((NG, 128), jnp.float32),
            pltpu.VMEM((NG, 1), jnp.float32),
        ],
    )(s3, s3, u3, dinv, b3, batch2, Wlin, blin)


def kernel(x, edge_index, batch, W1, b1, W2, b2, W3, b3, Wlin, blin):
    E = edge_index.shape[1]
    pad = EPAD - E
    src = edge_index[0]
    dst = edge_index[1]
    pad_src = (jnp.arange(pad, dtype=jnp.int32) * 37) % jnp.int32(N - 16)
    pad_dst = jnp.int32(N) + (jnp.arange(pad, dtype=jnp.int32) % 8)
    src_r = jnp.concatenate([src, pad_src]).reshape(ROWS, SW)
    dst_r = jnp.concatenate([dst, pad_dst]).reshape(ROWS, SW)

    zeros128 = jnp.zeros((N, 128), jnp.float32)
    zeros16 = jnp.zeros((N, 16), jnp.float32)
    ones16 = jnp.ones((N, 16), jnp.float32)

    b1r = b1.reshape(1, 256)
    b2r = b2.reshape(1, 256)
    b3r = b3.reshape(1, 128)
    blinr = blin.reshape(1, 2)
    batch2 = batch.reshape(N, 1)

    degp = _spmm_sc(ones16, src_r, dst_r, zeros16, 16, "deg")
    dinv, u1 = _tc_prep(degp, x)
    s1 = _spmm_sc(u1, src_r, dst_r, zeros128, 128, "edge")
    u2 = _tc_layer1(s1, u1, dinv, W1, b1r, W2)
    s2 = _spmm_sc(u2, src_r, dst_r, zeros128, 128, "feat")
    u3 = _tc_layer2(s2, u2, dinv, b2r, W3)
    s3 = _spmm_sc(u3, src_r, dst_r, zeros128, 128, "edge")
    return _tc_final(s3, u3, dinv, b3r, batch2, Wlin, blinr)


# speedup vs baseline: 24.9944x; 1.0725x over previous
import functools

import jax
import jax.numpy as jnp
from jax import lax
from jax.experimental import pallas as pl
from jax.experimental.pallas import tpu as pltpu
from jax.experimental.pallas import tpu_sc as plsc

N = 10000
NG = 64
NC = 2
NS = 16
NACC = N + 8
RB = 624
RBW = 640
SW = 64
SPS = 16
EPAD = 327680
ROWS = EPAD // SW
BN = 1000
GB = N // BN


def _spmm_sc(u, src_r, dst_r, zeros, d, mode):
    mesh = plsc.VectorSubcoreMesh(core_axis_name="c", subcore_axis_name="s")
    slabs_per_tile = (ROWS // NS if mode == "feat" else ROWS // (NC * NS)) // SPS
    n_pairs = slabs_per_tile // 2
    NB = 4
    LAG = 2

    def body(u_hbm, src_hbm, dst_hbm, zeros_hbm, out_hbm,
             idx_sa, idx_sb, idx_da, idx_db, rows, acc, s0, s1, s2, s3, isem):
        c = lax.axis_index("c")
        s = lax.axis_index("s")
        sems = [s0, s1, s2, s3]
        idx_s = idx_sa
        if mode == "deg":
            onev = jnp.ones((16,), jnp.float32)

            def fill(i, carry):
                for t in range(d // 16):
                    rows[0, i, pl.ds(t * 16, 16)] = onev
                return carry

            lax.fori_loop(0, SW, fill, 0)
        zstart = pl.multiple_of(s * RB, 8)
        pltpu.sync_copy(zeros_hbm.at[pl.ds(zstart, RBW)],
                        acc.at[pl.ds(zstart, RBW)])
        plsc.subcore_barrier()

        if mode == "feat":
            row0 = s * slabs_per_tile * SPS
            off = lax.broadcast(c * jnp.int32(N), (16,))
        else:
            row0 = (c * NS + s) * slabs_per_tile * SPS

        def fire_idx(r, idx_sx, idx_dx):
            pltpu.async_copy(src_hbm.at[pl.ds(r, SPS)], idx_sx, isem)
            pltpu.async_copy(dst_hbm.at[pl.ds(r, SPS)], idx_dx, isem)

        def wait_idx(idx_sx, idx_dx):
            pltpu.make_async_copy(src_hbm.at[pl.ds(0, SPS)], idx_sx,
                                  isem).wait()
            pltpu.make_async_copy(dst_hbm.at[pl.ds(0, SPS)], idx_dx,
                                  isem).wait()
            if mode == "feat":
                for j in range(SPS):
                    for t in range(SW // 16):
                        idx_sx[j, pl.ds(t * 16, 16)] = (
                            idx_sx[j, pl.ds(t * 16, 16)] + off)

        def scat(idx_d, b, buf):
            return pltpu.async_copy(rows.at[buf], acc.at[idx_d.at[b]],
                                    sems[buf], add=True)

        if mode == "deg":
            def pair(g2, carry):
                for idx_d, dr in ((idx_da, 0), (idx_db, SPS)):
                    r = pl.multiple_of(row0 + g2 * 2 * SPS + dr, 8)
                    pltpu.sync_copy(dst_hbm.at[pl.ds(r, SPS)], idx_d)
                    sds = [scat(idx_d, b, 0) for b in range(SPS)]
                    for sd in sds:
                        sd.wait()
                return carry

            lax.fori_loop(0, n_pairs, pair, 0)
        else:
            def half_run(idx_sx, idx_d, idx_d_prev, first, g2, hook):
                gds = [None] * SPS
                sds = [None] * SPS
                for b in range(SPS):
                    buf = b % NB
                    if b < NB:
                        if first:
                            @pl.when(g2 > 0)
                            def _(b=b, buf=buf):
                                pltpu.make_async_copy(
                                    rows.at[buf],
                                    acc.at[idx_d_prev.at[SPS - NB + b]],
                                    sems[buf]).wait()
                        else:
                            idx_d_prev[SPS - NB + b].wait()
                    else:
                        sds[b - NB].wait()
                    if b == NB:
                        hook()
                    gds[b] = pltpu.async_copy(u_hbm.at[idx_sx.at[b]],
                                              rows.at[buf], sems[buf])
                    if b >= LAG:
                        gds[b - LAG].wait()
                        sds[b - LAG] = scat(idx_d, b - LAG, (b - LAG) % NB)
                for b in range(SPS - LAG, SPS):
                    gds[b].wait()
                    sds[b] = scat(idx_d, b, b % NB)
                return sds

            fire_idx(pl.multiple_of(row0, 8), idx_sa, idx_da)

            def pair(g2, carry):
                r = pl.multiple_of(row0 + g2 * 2 * SPS, 8)
                wait_idx(idx_sa, idx_da)

                def hook_b():
                    fire_idx(r + SPS, idx_sb, idx_db)

                sds_a = half_run(idx_sa, idx_da, idx_db, True, g2, hook_b)
                wait_idx(idx_sb, idx_db)

                def hook_a():
                    @pl.when(g2 < n_pairs - 1)
                    def _():
                        fire_idx(r + 2 * SPS, idx_sa, idx_da)

                half_run(idx_sb, idx_db, sds_a, False, g2, hook_a)
                return carry

            lax.fori_loop(0, n_pairs, pair, 0)
            for b in range(SPS - NB, SPS):
                pltpu.make_async_copy(rows.at[b % NB],
                                      acc.at[idx_db.at[b]],
                                      sems[b % NB]).wait()

        plsc.subcore_barrier()
        rstart = pl.multiple_of(s * RB, 8)
        pltpu.sync_copy(acc.at[pl.ds(rstart, RBW)],
                        out_hbm.at[pl.ds(c * N + rstart, RBW)])

    return pl.kernel(
        body,
        out_type=jax.ShapeDtypeStruct((NC * N, d), jnp.float32),
        mesh=mesh,
        scratch_types=[
            pltpu.VMEM((SPS, SW), jnp.int32),
            pltpu.VMEM((SPS, SW), jnp.int32),
            pltpu.VMEM((SPS, SW), jnp.int32),
            pltpu.VMEM((SPS, SW), jnp.int32),
            pltpu.VMEM((NB, SW, d), jnp.float32),
            pltpu.VMEM_SHARED((NACC, d), jnp.float32),
            pltpu.SemaphoreType.DMA,
            pltpu.SemaphoreType.DMA,
            pltpu.SemaphoreType.DMA,
            pltpu.SemaphoreType.DMA,
            pltpu.SemaphoreType.DMA,
        ],
    )(u, src_r, dst_r, zeros)


def _tc_prep(degp, x):
    def body(dega, degb, x_ref, dinv_ref, u1_ref):
        d = dega[...][:, 0:1] + degb[...][:, 0:1] + 1.0
        dinv = lax.rsqrt(d)
        dinv_ref[...] = dinv
        u1_ref[...] = x_ref[...] * dinv

    return pl.pallas_call(
        body,
        grid=(GB,),
        in_specs=[
            pl.BlockSpec((BN, 16), lambda i: (i, 0)),
            pl.BlockSpec((BN, 16), lambda i: (GB + i, 0)),
            pl.BlockSpec((BN, 128), lambda i: (i, 0)),
        ],
        out_specs=[
            pl.BlockSpec((BN, 1), lambda i: (i, 0)),
            pl.BlockSpec((BN, 128), lambda i: (i, 0)),
        ],
        out_shape=[
            jax.ShapeDtypeStruct((N, 1), jnp.float32),
            jax.ShapeDtypeStruct((N, 128), jnp.float32),
        ],
    )(degp, degp, x)


def _tc_layer1(s1, u1, dinv, W1, b1, W2):
    def body(s1a, s1b, u1_ref, dinv_ref, W1_ref, b1_ref, W2_ref, u2_ref, h1_s):
        k = pl.program_id(1)

        @pl.when(k == 0)
        def _():
            z1 = dinv_ref[...] * (s1a[...] + s1b[...] + u1_ref[...])
            h1_s[...] = jnp.maximum(
                jnp.dot(z1, W1_ref[...], preferred_element_type=jnp.float32)
                + b1_ref[...], 0.0)

        u2_ref[...] = dinv_ref[...] * jnp.dot(
            h1_s[...], W2_ref[...], preferred_element_type=jnp.float32)

    return pl.pallas_call(
        body,
        grid=(GB, 2),
        in_specs=[
            pl.BlockSpec((BN, 128), lambda i, k: (i, 0)),
            pl.BlockSpec((BN, 128), lambda i, k: (GB + i, 0)),
            pl.BlockSpec((BN, 128), lambda i, k: (i, 0)),
            pl.BlockSpec((BN, 1), lambda i, k: (i, 0)),
            pl.BlockSpec((128, 256), lambda i, k: (0, 0)),
            pl.BlockSpec((1, 256), lambda i, k: (0, 0)),
            pl.BlockSpec((256, 128), lambda i, k: (0, k)),
        ],
        out_specs=pl.BlockSpec((BN, 128), lambda i, k: (k * GB + i, 0)),
        out_shape=jax.ShapeDtypeStruct((NC * N, 128), jnp.float32),
        scratch_shapes=[pltpu.VMEM((BN, 256), jnp.float32)],
    )(s1, s1, u1, dinv, W1, b1, W2)


def _tc_layer2(s2, u2, dinv, b2, W3):
    def body(s2a, s2b, u2a, u2b, dinv_ref, b2_ref, W3_ref, u3_ref):
        dv = dinv_ref[...]
        h2a = jnp.maximum(dv * (s2a[...] + u2a[...]) + b2_ref[...][:, 0:128],
                          0.0)
        h2b = jnp.maximum(dv * (s2b[...] + u2b[...]) + b2_ref[...][:, 128:256],
                          0.0)
        u3_ref[...] = dv * (
            jnp.dot(h2a, W3_ref[0:128, :], preferred_element_type=jnp.float32)
            + jnp.dot(h2b, W3_ref[128:256, :],
                      preferred_element_type=jnp.float32))

    return pl.pallas_call(
        body,
        grid=(GB,),
        in_specs=[
            pl.BlockSpec((BN, 128), lambda i: (i, 0)),
            pl.BlockSpec((BN, 128), lambda i: (GB + i, 0)),
            pl.BlockSpec((BN, 128), lambda i: (i, 0)),
            pl.BlockSpec((BN, 128), lambda i: (GB + i, 0)),
            pl.BlockSpec((BN, 1), lambda i: (i, 0)),
            pl.BlockSpec((1, 256), lambda i: (0, 0)),
            pl.BlockSpec((256, 128), lambda i: (0, 0)),
        ],
        out_specs=pl.BlockSpec((BN, 128), lambda i: (i, 0)),
        out_shape=jax.ShapeDtypeStruct((N, 128), jnp.float32),
    )(s2, s2, u2, u2, dinv, b2, W3)


def _tc_final(s3, u3, dinv, b3, batch2, Wlin, blin):
    def body(s3a, s3b, u3_ref, dinv_ref, b3_ref, batch_ref, Wlin_ref,
             blin_ref, out_ref, sums, cnts):
        i = pl.program_id(0)
        z3 = dinv_ref[...] * (s3a[...] + s3b[...] + u3_ref[...]) + b3_ref[...]
        g = batch_ref[...]
        iota = lax.broadcasted_iota(jnp.int32, (BN, NG), 1)
        oh = (g == iota).astype(jnp.float32)
        part = lax.dot_general(oh, z3, (((0,), (0,)), ((), ())),
                               preferred_element_type=jnp.float32)
        cnt = jnp.sum(oh, axis=0).reshape(NG, 1)

        @pl.when(i == 0)
        def _():
            sums[...] = part
            cnts[...] = cnt

        @pl.when(i > 0)
        def _():
            sums[...] += part
            cnts[...] += cnt

        @pl.when(i == GB - 1)
        def _():
            pooled = sums[...] / jnp.maximum(cnts[...], 1.0)
            out_ref[...] = jnp.dot(pooled, Wlin_ref[...],
                                   preferred_element_type=jnp.float32
                                   ) + blin_ref[...]

    return pl.pallas_call(
        body,
        grid=(GB,),
        in_specs=[
            pl.BlockSpec((BN, 128), lambda i: (i, 0)),
            pl.BlockSpec((BN, 128), lambda i: (GB + i, 0)),
            pl.BlockSpec((BN, 128), lambda i: (i, 0)),
            pl.BlockSpec((BN, 1), lambda i: (i, 0)),
            pl.BlockSpec((1, 128), lambda i: (0, 0)),
            pl.BlockSpec((BN, 1), lambda i: (i, 0)),
            pl.BlockSpec((128, 2), lambda i: (0, 0)),
            pl.BlockSpec((1, 2), lambda i: (0, 0)),
        ],
        out_specs=pl.BlockSpec((NG, 2), lambda i: (0, 0)),
        out_shape=jax.ShapeDtypeStruct((NG, 2), jnp.float32),
        scratch_shapes=[
            pltpu.VMEM((NG, 128), jnp.float32),
            pltpu.VMEM((NG, 1), jnp.float32),
        ],
    )(s3, s3, u3, dinv, b3, batch2, Wlin, blin)


def kernel(x, edge_index, batch, W1, b1, W2, b2, W3, b3, Wlin, blin):
    E = edge_index.shape[1]
    pad = EPAD - E
    src = edge_index[0]
    dst = edge_index[1]
    pad_src = (jnp.arange(pad, dtype=jnp.int32) * 37) % jnp.int32(N - 16)
    pad_dst = jnp.int32(N) + (jnp.arange(pad, dtype=jnp.int32) % 8)
    src_r = jnp.concatenate([src, pad_src]).reshape(ROWS, SW)
    dst_r = jnp.concatenate([dst, pad_dst]).reshape(ROWS, SW)

    zeros128 = jnp.zeros((N, 128), jnp.float32)
    zeros16 = jnp.zeros((N, 16), jnp.float32)
    ones16 = jnp.ones((N, 16), jnp.float32)

    b1r = b1.reshape(1, 256)
    b2r = b2.reshape(1, 256)
    b3r = b3.reshape(1, 128)
    blinr = blin.reshape(1, 2)
    batch2 = batch.reshape(N, 1)

    degp = _spmm_sc(ones16, src_r, dst_r, zeros16, 16, "deg")
    dinv, u1 = _tc_prep(degp, x)
    s1 = _spmm_sc(u1, src_r, dst_r, zeros128, 128, "edge")
    u2 = _tc_layer1(s1, u1, dinv, W1, b1r, W2)
    s2 = _spmm_sc(u2, src_r, dst_r, zeros128, 128, "feat")
    u3 = _tc_layer2(s2, u2, dinv, b2r, W3)
    s3 = _spmm_sc(u3, src_r, dst_r, zeros128, 128, "edge")
    return _tc_final(s3, u3, dinv, b3r, batch2, Wlin, blinr)
